# baseline (speedup 1.0000x reference)
import functools

import jax
import jax.numpy as jnp
from jax import lax
from jax.experimental import pallas as pl
from jax.experimental.pallas import tpu as pltpu
from jax.experimental.pallas import tpu_sc as plsc

B, T, K = 128, 8192, 16
N = B * T
NC, NS, L = 2, 16, 16
NW = NC * NS
R = 1024
BS = 40
NTC = BS * T
CPW = (N - NTC) // NW // R
NBUF = 2
TCT = 8192

_mesh = plsc.VectorSubcoreMesh(core_axis_name="c", subcore_axis_name="s")


@functools.partial(
    pl.kernel,
    mesh=_mesh,
    out_type=jax.ShapeDtypeStruct((N - NTC,), jnp.float32),
    compiler_params=pltpu.CompilerParams(needs_layout_passes=False),
    scratch_types=[
        pltpu.VMEM((NBUF, 2, 8, 8, 128), jnp.float32),
        pltpu.VMEM((NBUF, 2, 8, 8, 128), jnp.float32),
        pltpu.VMEM((NBUF, 8, 128), jnp.int32),
        pltpu.VMEM((NBUF, R), jnp.float32),
        pltpu.VMEM((NBUF, R), jnp.float32),
        pltpu.SemaphoreType.DMA,
        pltpu.SemaphoreType.DMA,
        pltpu.SemaphoreType.DMA,
        pltpu.SemaphoreType.DMA,
    ],
)
def _sc_sample(mus_x, sig_x, z_x, noise_x, out_hbm,
               mu_v, sg_v, z_v, nz_v, out_v,
               in_sem0, in_sem1, out_sem0, out_sem1):
    wid = lax.axis_index("s") * NC + lax.axis_index("c")
    cc0 = BS * 8 + wid * CPW

    in_sems = (in_sem0, in_sem1)
    out_sems = (out_sem0, out_sem1)

    def chunk_copies(cc, p, sem):
        b = cc // 8
        tt0 = (cc % 8) * 8
        n0 = b * 128 + tt0
        zn0 = (b // 8) * 64 + tt0
        zbs = b % 8
        return (
            pltpu.make_async_copy(mus_x.at[pl.ds(n0, 8)], mu_v.at[p, 0], sem),
            pltpu.make_async_copy(mus_x.at[pl.ds(n0 + 64, 8)], mu_v.at[p, 1], sem),
            pltpu.make_async_copy(sig_x.at[pl.ds(n0, 8)], sg_v.at[p, 0], sem),
            pltpu.make_async_copy(sig_x.at[pl.ds(n0 + 64, 8)], sg_v.at[p, 1], sem),
            pltpu.make_async_copy(z_x.at[pl.ds(zn0, 8), zbs], z_v.at[p], sem),
            pltpu.make_async_copy(noise_x.at[pl.ds(cc * R, R)], nz_v.at[p], sem),
        )

    def store_copy(cc, p, sem):
        return pltpu.make_async_copy(
            out_v.at[p], out_hbm.at[pl.ds(cc * R - NTC, R)], sem)

    def start_loads(cc, p):
        for c in chunk_copies(cc, p, in_sems[p]):
            c.start()

    def wait_loads(cc, p):
        for c in chunk_copies(cc, p, in_sems[p]):
            c.wait()

    def compute(p):
        def vec_body(i, carry):
            tv = lax.iota(jnp.int32, L) + i * L
            zv = z_v[p, i // 8, pl.ds((i % 8) * L, L)]
            nv = nz_v[p, pl.ds(i * L, L)]
            khi = zv >> 3
            ks = zv & 7
            tt = tv >> 7
            tl = tv & 127
            mu = plsc.load_gather(mu_v.at[p], [khi, tt, ks, tl])
            sg = plsc.load_gather(sg_v.at[p], [khi, tt, ks, tl])
            out_v[p, pl.ds(i * L, L)] = mu + sg * nv
            return carry

        lax.fori_loop(0, R // L, vec_body, 0, unroll=8)

    def half_step(g, c, p):
        cc = cc0 + c
        nxt = jnp.minimum(c + 1, CPW - 1)
        start_loads(cc0 + nxt, 1 - p)
        wait_loads(cc, p)

        @pl.when(g > 0)
        def _():
            store_copy(cc - 2, p, out_sems[p]).wait()

        compute(p)
        store_copy(cc, p, out_sems[p]).start()

    start_loads(cc0, 0)

    def pair_body(g, carry):
        half_step(g, 2 * g, 0)
        half_step(g, 2 * g + 1, 1)
        return carry

    lax.fori_loop(0, CPW // 2, pair_body, 0)

    wait_loads(cc0 + CPW - 1, 1 - (CPW - 1) % 2)
    store_copy(cc0 + CPW - 2, (CPW - 2) % 2, out_sems[(CPW - 2) % 2]).wait()
    store_copy(cc0 + CPW - 1, (CPW - 1) % 2, out_sems[(CPW - 1) % 2]).wait()


def _tc_body(mus_ref, sig_ref, z_ref, nz_ref, out_ref):
    k_iota = lax.broadcasted_iota(jnp.int32, (K, 128), 0)
    bs = pl.program_id(0) % 8
    for tt in range(T // 128):
        s = pl.ds(tt * 128, 128)
        msk = k_iota == z_ref[tt, bs, :][None, :]
        mu = jnp.sum(jnp.where(msk, mus_ref[0, :, s], 0.0), axis=0)
        sg = jnp.sum(jnp.where(msk, sig_ref[0, :, s], 0.0), axis=0)
        out_ref[s] = mu + sg * nz_ref[s]


_tc_sample = pl.pallas_call(
    _tc_body,
    grid=(BS,),
    in_specs=[
        pl.BlockSpec((1, K, T), lambda g: (g, 0, 0)),
        pl.BlockSpec((1, K, T), lambda g: (g, 0, 0)),
        pl.BlockSpec((64, 8, 128), lambda g: (g // 8, 0, 0)),
        pl.BlockSpec((T,), lambda g: (g,)),
    ],
    out_specs=pl.BlockSpec((T,), lambda g: (g,)),
    out_shape=jax.ShapeDtypeStruct((NTC,), jnp.float32),
)


def kernel(mus, sigmas, z, noise):
    mus_t = jnp.swapaxes(mus, 1, 2)
    sig_t = jnp.swapaxes(sigmas, 1, 2)
    mus_x = (mus.reshape(B, 64, 128, 2, 8)
             .transpose(0, 3, 1, 4, 2)
             .reshape(B * 2 * 64, 8, 128))
    sig_x = (sigmas.reshape(B, 64, 128, 2, 8)
             .transpose(0, 3, 1, 4, 2)
             .reshape(B * 2 * 64, 8, 128))
    z_x = (z.astype(jnp.int32)
           .reshape(16, 8, 64, 128)
           .transpose(0, 2, 1, 3)
           .reshape(1024, 8, 128))
    noise_x = noise.reshape(-1)
    sc_out = _sc_sample(mus_x, sig_x, z_x, noise_x)
    tc_out = _tc_sample(mus_t, sig_t, z_x, noise_x)
    out = jnp.concatenate([tc_out, sc_out])
    return out.reshape(B, T, 1)


# speedup vs baseline: 1.0291x; 1.0291x over previous
import functools

import jax
import jax.numpy as jnp
from jax import lax
from jax.experimental import pallas as pl
from jax.experimental.pallas import tpu as pltpu
from jax.experimental.pallas import tpu_sc as plsc

B, T, K = 128, 8192, 16
N = B * T
NC, NS, L = 2, 16, 16
NW = NC * NS
R = 1024
BS = 44
NTC = BS * T
CPW = (N - NTC) // NW // R
NBUF = 2
TCT = 8192

_mesh = plsc.VectorSubcoreMesh(core_axis_name="c", subcore_axis_name="s")


@functools.partial(
    pl.kernel,
    mesh=_mesh,
    out_type=jax.ShapeDtypeStruct((N - NTC,), jnp.float32),
    compiler_params=pltpu.CompilerParams(needs_layout_passes=False),
    scratch_types=[
        pltpu.VMEM((NBUF, 2, 8, 8, 128), jnp.float32),
        pltpu.VMEM((NBUF, 2, 8, 8, 128), jnp.float32),
        pltpu.VMEM((NBUF, 8, 128), jnp.int32),
        pltpu.VMEM((NBUF, R), jnp.float32),
        pltpu.VMEM((NBUF, R), jnp.float32),
        pltpu.SemaphoreType.DMA,
        pltpu.SemaphoreType.DMA,
        pltpu.SemaphoreType.DMA,
        pltpu.SemaphoreType.DMA,
    ],
)
def _sc_sample(mus_x, sig_x, z_x, noise_x, out_hbm,
               mu_v, sg_v, z_v, nz_v, out_v,
               in_sem0, in_sem1, out_sem0, out_sem1):
    wid = lax.axis_index("s") * NC + lax.axis_index("c")
    cc0 = BS * 8 + wid * CPW

    in_sems = (in_sem0, in_sem1)
    out_sems = (out_sem0, out_sem1)

    def chunk_copies(cc, p, sem):
        b = cc // 8
        tt0 = (cc % 8) * 8
        n0 = b * 128 + tt0
        zn0 = (b // 8) * 64 + tt0
        zbs = b % 8
        return (
            pltpu.make_async_copy(mus_x.at[pl.ds(n0, 8)], mu_v.at[p, 0], sem),
            pltpu.make_async_copy(mus_x.at[pl.ds(n0 + 64, 8)], mu_v.at[p, 1], sem),
            pltpu.make_async_copy(sig_x.at[pl.ds(n0, 8)], sg_v.at[p, 0], sem),
            pltpu.make_async_copy(sig_x.at[pl.ds(n0 + 64, 8)], sg_v.at[p, 1], sem),
            pltpu.make_async_copy(z_x.at[pl.ds(zn0, 8), zbs], z_v.at[p], sem),
            pltpu.make_async_copy(noise_x.at[pl.ds(cc * R, R)], nz_v.at[p], sem),
        )

    def store_copy(cc, p, sem):
        return pltpu.make_async_copy(
            out_v.at[p], out_hbm.at[pl.ds(cc * R - NTC, R)], sem)

    def start_loads(cc, p):
        for c in chunk_copies(cc, p, in_sems[p]):
            c.start()

    def wait_loads(cc, p):
        for c in chunk_copies(cc, p, in_sems[p]):
            c.wait()

    def compute(p):
        def vec_body(i, carry):
            tv = lax.iota(jnp.int32, L) + i * L
            zv = z_v[p, i // 8, pl.ds((i % 8) * L, L)]
            nv = nz_v[p, pl.ds(i * L, L)]
            khi = zv >> 3
            ks = zv & 7
            tt = tv >> 7
            tl = tv & 127
            mu = plsc.load_gather(mu_v.at[p], [khi, tt, ks, tl])
            sg = plsc.load_gather(sg_v.at[p], [khi, tt, ks, tl])
            out_v[p, pl.ds(i * L, L)] = mu + sg * nv
            return carry

        lax.fori_loop(0, R // L, vec_body, 0, unroll=8)

    def half_step(g, c, p):
        cc = cc0 + c
        nxt = jnp.minimum(c + 1, CPW - 1)
        start_loads(cc0 + nxt, 1 - p)
        wait_loads(cc, p)

        @pl.when(g > 0)
        def _():
            store_copy(cc - 2, p, out_sems[p]).wait()

        compute(p)
        store_copy(cc, p, out_sems[p]).start()

    start_loads(cc0, 0)

    def pair_body(g, carry):
        half_step(g, 2 * g, 0)
        half_step(g, 2 * g + 1, 1)
        return carry

    lax.fori_loop(0, CPW // 2, pair_body, 0)

    if CPW % 2:
        c = CPW - 1
        wait_loads(cc0 + c, 0)
        store_copy(cc0 + c - 2, 0, out_sems[0]).wait()
        compute(0)
        store_copy(cc0 + c, 0, out_sems[0]).start()
        store_copy(cc0 + c - 1, 1, out_sems[1]).wait()
        store_copy(cc0 + c, 0, out_sems[0]).wait()
    else:
        wait_loads(cc0 + CPW - 1, 1 - (CPW - 1) % 2)
        store_copy(cc0 + CPW - 2, (CPW - 2) % 2, out_sems[(CPW - 2) % 2]).wait()
        store_copy(cc0 + CPW - 1, (CPW - 1) % 2, out_sems[(CPW - 1) % 2]).wait()


def _tc_body(mus_ref, sig_ref, z_ref, nz_ref, out_ref):
    k_iota = lax.broadcasted_iota(jnp.int32, (K, 128), 0)
    bs = pl.program_id(0) % 8
    for tt in range(T // 128):
        s = pl.ds(tt * 128, 128)
        msk = k_iota == z_ref[tt, bs, :][None, :]
        mu = jnp.sum(jnp.where(msk, mus_ref[0, :, s], 0.0), axis=0)
        sg = jnp.sum(jnp.where(msk, sig_ref[0, :, s], 0.0), axis=0)
        out_ref[s] = mu + sg * nz_ref[s]


_tc_sample = pl.pallas_call(
    _tc_body,
    grid=(BS,),
    in_specs=[
        pl.BlockSpec((1, K, T), lambda g: (g, 0, 0)),
        pl.BlockSpec((1, K, T), lambda g: (g, 0, 0)),
        pl.BlockSpec((64, 8, 128), lambda g: (g // 8, 0, 0)),
        pl.BlockSpec((T,), lambda g: (g,)),
    ],
    out_specs=pl.BlockSpec((T,), lambda g: (g,)),
    out_shape=jax.ShapeDtypeStruct((NTC,), jnp.float32),
)


def kernel(mus, sigmas, z, noise):
    mus_t = jnp.swapaxes(mus, 1, 2)
    sig_t = jnp.swapaxes(sigmas, 1, 2)
    mus_x = (mus.reshape(B, 64, 128, 2, 8)
             .transpose(0, 3, 1, 4, 2)
             .reshape(B * 2 * 64, 8, 128))
    sig_x = (sigmas.reshape(B, 64, 128, 2, 8)
             .transpose(0, 3, 1, 4, 2)
             .reshape(B * 2 * 64, 8, 128))
    z_x = (z.astype(jnp.int32)
           .reshape(16, 8, 64, 128)
           .transpose(0, 2, 1, 3)
           .reshape(1024, 8, 128))
    noise_x = noise.reshape(-1)
    sc_out = _sc_sample(mus_x, sig_x, z_x, noise_x)
    tc_out = _tc_sample(mus_t, sig_t, z_x, noise_x)
    out = jnp.concatenate([tc_out, sc_out])
    return out.reshape(B, T, 1)
